# baseline (speedup 1.0000x reference)
import functools

import jax
import jax.numpy as jnp
from jax import lax
from jax.experimental import pallas as pl
from jax.experimental.pallas import tpu as pltpu
from jax.experimental.pallas import tpu_sc as plsc

_K = 2048
_F = 256
_N = 50000
_BLK = 512
_NP = 50176
_NB = _NP // _BLK
_CHUNK = 256
_NCH = _K // _CHUNK
_INT_MIN = -2147483648

_NW = 32
_BPW = _K // _NW


def _sortkey(x):
    i = lax.bitcast_convert_type(x, jnp.int32)
    return jnp.where(i < 0, i ^ jnp.int32(0x7FFFFFFF), i)


def _scores_body(emb_ref, scorer_ref, nrm_ref, out_ref):
    b = pl.program_id(0)
    w = scorer_ref[...]
    s = jnp.dot(emb_ref[...], w, preferred_element_type=jnp.float32)
    s = s / nrm_ref[0, 0]
    row = b * _BLK + lax.broadcasted_iota(jnp.int32, (_BLK, 1), 0)
    out_ref[...] = jnp.where(row < _N, s, -jnp.inf).reshape(1, _BLK, 1)


def _scores_call(embs_p, scorer):
    return pl.pallas_call(
        _scores_body,
        grid=(_NB,),
        in_specs=[
            pl.BlockSpec((_BLK, _F), lambda b: (b, 0)),
            pl.BlockSpec((_F, 1), lambda b: (0, 0)),
            pl.BlockSpec((1, 1), lambda b: (0, 0)),
        ],
        out_specs=pl.BlockSpec((1, _BLK, 1), lambda b: (b, 0, 0)),
        out_shape=jax.ShapeDtypeStruct((_NB, _BLK, 1), jnp.float32),
    )(embs_p, scorer, jnp.linalg.norm(scorer).reshape(1, 1))


def _thresh_body(sc_ref, out_ref):
    skey = _sortkey(sc_ref[...])

    def step(t, ans_u):
        bbit = 31 - t
        trial_u = ans_u | (jnp.int32(1) << bbit)
        trial_s = trial_u ^ _INT_MIN
        cnt = jnp.sum((skey >= trial_s).astype(jnp.int32))
        return jnp.where(cnt >= _K, trial_u, ans_u)

    ans_u = lax.fori_loop(0, 32, step, jnp.int32(0))
    kth = ans_u ^ _INT_MIN
    cnt_gt = jnp.sum((skey > kth).astype(jnp.int32))
    ties = _K - cnt_gt
    col = lax.broadcasted_iota(jnp.int32, (8, 128), 1)
    out_ref[...] = jnp.where(col == 0, kth, jnp.where(col == 1, ties, 0))


def _thresh_call(scores):
    return pl.pallas_call(
        _thresh_body,
        out_shape=jax.ShapeDtypeStruct((8, 128), jnp.int32),
    )(scores)


def _compact_body(sc_ref, th_ref, cand_ref, cnt_ref):
    b = pl.program_id(0)

    @pl.when(b == 0)
    def _():
        cnt_ref[0] = 0
        cnt_ref[1] = 0
        cand_ref[...] = jnp.zeros((2, _K), jnp.float32)

    kth = th_ref[0, 0]
    ties = th_ref[0, 1]
    s = sc_ref[...].reshape(_BLK, 1)
    key = _sortkey(s)
    gt = key > kth
    eq = key == kth
    c_tot = cnt_ref[0]
    c_eq = cnt_ref[1]

    tri = (lax.broadcasted_iota(jnp.int32, (_BLK, _BLK), 1)
           < lax.broadcasted_iota(jnp.int32, (_BLK, _BLK), 0)).astype(jnp.float32)
    eq_f = eq.astype(jnp.float32)
    eqcum = jnp.dot(tri, eq_f, preferred_element_type=jnp.float32,
                    precision=lax.Precision.HIGHEST)
    take_eq = eq & ((c_eq + eqcum.astype(jnp.int32)) < ties)
    sel = gt | take_eq
    sel_f = sel.astype(jnp.float32)
    selcum = jnp.dot(tri, sel_f, preferred_element_type=jnp.float32,
                     precision=lax.Precision.HIGHEST)
    pos = c_tot + selcum.astype(jnp.int32)

    iota_p = lax.broadcasted_iota(jnp.int32, (_BLK, _K), 1)
    S = jnp.where((iota_p == pos) & sel, 1.0, 0.0)

    idx = (b * _BLK
           + lax.broadcasted_iota(jnp.int32, (_BLK, 1), 0)).astype(jnp.float32)
    s_fin = jnp.maximum(s, jnp.float32(-3.0e38))
    x = jnp.concatenate([s_fin, idx], axis=1)
    cand_ref[...] += lax.dot_general(x, S, (((0,), (0,)), ((), ())),
                                     preferred_element_type=jnp.float32,
                                     precision=lax.Precision.HIGHEST)

    cnt_ref[0] = c_tot + jnp.sum(sel.astype(jnp.int32))
    cnt_ref[1] = c_eq + jnp.sum(eq.astype(jnp.int32))


def _compact_call(scores, thresh):
    return pl.pallas_call(
        _compact_body,
        grid=(_NB,),
        in_specs=[
            pl.BlockSpec((1, _BLK, 1), lambda b: (b, 0, 0)),
            pl.BlockSpec((8, 128), lambda b: (0, 0)),
        ],
        out_specs=pl.BlockSpec((2, _K), lambda b: (0, 0)),
        out_shape=jax.ShapeDtypeStruct((2, _K), jnp.float32),
        scratch_shapes=[pltpu.SMEM((2,), jnp.int32)],
    )(scores, thresh)


def _sc_gather(table, idx):
    mesh = plsc.VectorSubcoreMesh(core_axis_name="c", subcore_axis_name="s")

    @functools.partial(
        pl.kernel,
        mesh=mesh,
        out_type=jax.ShapeDtypeStruct((_K, _F), jnp.float32),
        scratch_types=[
            pltpu.VMEM((_BPW,), jnp.int32),
            pltpu.VMEM((_BPW, _F), jnp.float32),
            pltpu.SemaphoreType.DMA,
        ],
    )
    def k(table_hbm, idx_hbm, out_hbm, idx_v, rows_v, sem):
        wid = lax.axis_index("s") * 2 + lax.axis_index("c")
        base = wid * _BPW
        pltpu.sync_copy(idx_hbm.at[pl.ds(base, _BPW)], idx_v)
        pltpu.async_copy(table_hbm.at[idx_v], rows_v, sem).wait()
        pltpu.sync_copy(rows_v, out_hbm.at[pl.ds(base, _BPW)])

    return k(table, idx)


def _finalize_body(candf_ref, candc_ref, g_ref, out_ref):
    b = pl.program_id(0)

    @pl.when(b == 0)
    def _():
        out_ref[...] = jnp.zeros((_F, _K), jnp.float32)

    keys_all = _sortkey(candf_ref[0:1, :])
    idx_all = candf_ref[1:2, :]

    eye = (lax.broadcasted_iota(jnp.int32, (_CHUNK, _CHUNK), 0)
           == lax.broadcasted_iota(jnp.int32, (_CHUNK, _CHUNK), 1)
           ).astype(jnp.float32)
    cc = lax.dot_general(eye, candc_ref[...], (((1,), (1,)), ((), ())),
                         preferred_element_type=jnp.float32,
                         precision=lax.Precision.HIGHEST)
    score_col = cc[:, 0:1]
    idx_col = cc[:, 1:2]
    key_col = _sortkey(score_col)

    gtm = (keys_all > key_col).astype(jnp.int32)
    eqm = ((keys_all == key_col) & (idx_all < idx_col)).astype(jnp.int32)
    rank = jnp.sum(gtm + eqm, axis=1, keepdims=True)

    iota_r = lax.broadcasted_iota(jnp.int32, (_CHUNK, _K), 1)
    P = (iota_r == rank).astype(jnp.float32)

    gs = g_ref[...] * jnp.tanh(score_col)
    out_ref[...] += lax.dot_general(gs, P, (((0,), (0,)), ((), ())),
                                    preferred_element_type=jnp.float32,
                                    precision=lax.Precision.HIGHEST)


def _finalize_call(cand, g):
    return pl.pallas_call(
        _finalize_body,
        grid=(_NCH,),
        in_specs=[
            pl.BlockSpec((2, _K), lambda b: (0, 0)),
            pl.BlockSpec((2, _CHUNK), lambda b: (0, b)),
            pl.BlockSpec((_CHUNK, _F), lambda b: (b, 0)),
        ],
        out_specs=pl.BlockSpec((_F, _K), lambda b: (0, 0)),
        out_shape=jax.ShapeDtypeStruct((_F, _K), jnp.float32),
    )(cand, cand, g)


def kernel(node_embs, scorer):
    embs_p = jnp.pad(node_embs, ((0, _NP - _N), (0, 0)))
    scores = _scores_call(embs_p, scorer)
    thresh = _thresh_call(scores.reshape(_NP // 128, 128))
    cand = _compact_call(scores, thresh)
    idx = cand[1].astype(jnp.int32)
    g = _sc_gather(node_embs, idx)
    return _finalize_call(cand, g)


# speedup vs baseline: 1.6206x; 1.6206x over previous
import functools

import jax
import jax.numpy as jnp
from jax import lax
from jax.experimental import pallas as pl
from jax.experimental.pallas import tpu as pltpu
from jax.experimental.pallas import tpu_sc as plsc

_K = 2048
_F = 256
_N = 50000
_BLK = 512
_NP = 50176
_NB = _NP // _BLK
_CHUNK = 256
_NCH = _K // _CHUNK
_INT_MIN = -2147483648

_NW = 32
_BPW = _K // _NW


def _sortkey(x):
    i = lax.bitcast_convert_type(x, jnp.int32)
    return jnp.where(i < 0, i ^ jnp.int32(0x7FFFFFFF), i)


def _scores_body(emb_ref, scorer_ref, nrm_ref, out_ref):
    b = pl.program_id(0)
    w = scorer_ref[...]
    s = jnp.dot(emb_ref[...], w, preferred_element_type=jnp.float32)
    s = s / nrm_ref[0, 0]
    row = b * _BLK + lax.broadcasted_iota(jnp.int32, (_BLK, 1), 0)
    out_ref[...] = jnp.where(row < _N, s, -jnp.inf).reshape(1, _BLK, 1)


def _scores_call(embs_p, scorer):
    return pl.pallas_call(
        _scores_body,
        grid=(_NB,),
        in_specs=[
            pl.BlockSpec((_BLK, _F), lambda b: (b, 0)),
            pl.BlockSpec((_F, 1), lambda b: (0, 0)),
            pl.BlockSpec((1, 1), lambda b: (0, 0)),
        ],
        out_specs=pl.BlockSpec((1, _BLK, 1), lambda b: (b, 0, 0)),
        out_shape=jax.ShapeDtypeStruct((_NB, _BLK, 1), jnp.float32),
    )(embs_p, scorer, jnp.linalg.norm(scorer).reshape(1, 1))


def _thresh_body(sc_ref, out_ref):
    skey = _sortkey(sc_ref[...])

    def step(t, ans_u):
        bbit = 31 - t
        trial_u = ans_u | (jnp.int32(1) << bbit)
        trial_s = trial_u ^ _INT_MIN
        cnt = jnp.sum((skey >= trial_s).astype(jnp.int32))
        return jnp.where(cnt >= _K, trial_u, ans_u)

    ans_u = lax.fori_loop(0, 32, step, jnp.int32(0))
    kth = ans_u ^ _INT_MIN
    cnt_gt = jnp.sum((skey > kth).astype(jnp.int32))
    ties = _K - cnt_gt
    col = lax.broadcasted_iota(jnp.int32, (8, 128), 1)
    out_ref[...] = jnp.where(col == 0, kth, jnp.where(col == 1, ties, 0))


def _thresh_call(scores):
    return pl.pallas_call(
        _thresh_body,
        out_shape=jax.ShapeDtypeStruct((8, 128), jnp.int32),
    )(scores)


def _compact_body(sc_ref, th_ref, cand_ref, cnt_ref):
    b = pl.program_id(0)

    @pl.when(b == 0)
    def _():
        cnt_ref[0] = 0
        cnt_ref[1] = 0
        cand_ref[...] = jnp.zeros((2, _K), jnp.float32)

    kth = th_ref[0, 0]
    ties = th_ref[0, 1]
    s = sc_ref[...].reshape(_BLK, 1)
    key = _sortkey(s)
    gt = key > kth
    eq = key == kth
    c_tot = cnt_ref[0]
    c_eq = cnt_ref[1]

    tri = (lax.broadcasted_iota(jnp.int32, (_BLK, _BLK), 1)
           < lax.broadcasted_iota(jnp.int32, (_BLK, _BLK), 0)).astype(jnp.float32)
    ge = jnp.concatenate([gt.astype(jnp.float32), eq.astype(jnp.float32)],
                         axis=1)
    cums = jnp.dot(tri, ge, preferred_element_type=jnp.float32)
    gtcum = cums[:, 0:1].astype(jnp.int32)
    eqcum = cums[:, 1:2].astype(jnp.int32)
    take_eq = eq & ((c_eq + eqcum) < ties)
    sel = gt | take_eq
    takecum = jnp.minimum(eqcum, jnp.maximum(ties - c_eq, 0))
    pos = c_tot + gtcum + takecum

    iota_p = lax.broadcasted_iota(jnp.int32, (_BLK, _K), 1)
    S = jnp.where((iota_p == pos) & sel, 1.0, 0.0)

    idx = (b * _BLK
           + lax.broadcasted_iota(jnp.int32, (_BLK, 1), 0)).astype(jnp.float32)
    s_fin = jnp.maximum(s, jnp.float32(-3.0e38))
    x = jnp.concatenate([s_fin, idx], axis=1)
    cand_ref[...] += lax.dot_general(x, S, (((0,), (0,)), ((), ())),
                                     preferred_element_type=jnp.float32,
                                     precision=lax.Precision.HIGHEST)

    cnt_ref[0] = c_tot + jnp.sum(sel.astype(jnp.int32))
    cnt_ref[1] = c_eq + jnp.sum(eq.astype(jnp.int32))


def _compact_call(scores, thresh):
    return pl.pallas_call(
        _compact_body,
        grid=(_NB,),
        in_specs=[
            pl.BlockSpec((1, _BLK, 1), lambda b: (b, 0, 0)),
            pl.BlockSpec((8, 128), lambda b: (0, 0)),
        ],
        out_specs=pl.BlockSpec((2, _K), lambda b: (0, 0)),
        out_shape=jax.ShapeDtypeStruct((2, _K), jnp.float32),
        scratch_shapes=[pltpu.SMEM((2,), jnp.int32)],
    )(scores, thresh)


def _sc_gather(table, idx):
    mesh = plsc.VectorSubcoreMesh(core_axis_name="c", subcore_axis_name="s")

    @functools.partial(
        pl.kernel,
        mesh=mesh,
        out_type=jax.ShapeDtypeStruct((_K, _F), jnp.float32),
        scratch_types=[
            pltpu.VMEM((_BPW,), jnp.int32),
            pltpu.VMEM((_BPW, _F), jnp.float32),
            pltpu.SemaphoreType.DMA,
        ],
    )
    def k(table_hbm, idx_hbm, out_hbm, idx_v, rows_v, sem):
        wid = lax.axis_index("s") * 2 + lax.axis_index("c")
        base = wid * _BPW
        pltpu.sync_copy(idx_hbm.at[pl.ds(base, _BPW)], idx_v)
        pltpu.async_copy(table_hbm.at[idx_v], rows_v, sem).wait()
        pltpu.sync_copy(rows_v, out_hbm.at[pl.ds(base, _BPW)])

    return k(table, idx)


def _finalize_body(candf_ref, candc_ref, g_ref, out_ref):
    b = pl.program_id(0)

    @pl.when(b == 0)
    def _():
        out_ref[...] = jnp.zeros((_F, _K), jnp.float32)

    keys_all = _sortkey(candf_ref[0:1, :])
    idx_all = candf_ref[1:2, :]

    eye = (lax.broadcasted_iota(jnp.int32, (_CHUNK, _CHUNK), 0)
           == lax.broadcasted_iota(jnp.int32, (_CHUNK, _CHUNK), 1)
           ).astype(jnp.float32)
    cc = lax.dot_general(eye, candc_ref[...], (((1,), (1,)), ((), ())),
                         preferred_element_type=jnp.float32,
                         precision=lax.Precision.HIGHEST)
    score_col = cc[:, 0:1]
    idx_col = cc[:, 1:2]
    key_col = _sortkey(score_col)

    gtm = (keys_all > key_col).astype(jnp.int32)
    eqm = ((keys_all == key_col) & (idx_all < idx_col)).astype(jnp.int32)
    rank = jnp.sum(gtm + eqm, axis=1, keepdims=True)

    iota_r = lax.broadcasted_iota(jnp.int32, (_CHUNK, _K), 1)
    P = (iota_r == rank).astype(jnp.float32)

    gs = g_ref[...] * jnp.tanh(score_col)
    out_ref[...] += lax.dot_general(gs, P, (((0,), (0,)), ((), ())),
                                    preferred_element_type=jnp.float32,
                                    precision=lax.Precision.HIGHEST)


def _finalize_call(cand, g):
    return pl.pallas_call(
        _finalize_body,
        grid=(_NCH,),
        in_specs=[
            pl.BlockSpec((2, _K), lambda b: (0, 0)),
            pl.BlockSpec((2, _CHUNK), lambda b: (0, b)),
            pl.BlockSpec((_CHUNK, _F), lambda b: (b, 0)),
        ],
        out_specs=pl.BlockSpec((_F, _K), lambda b: (0, 0)),
        out_shape=jax.ShapeDtypeStruct((_F, _K), jnp.float32),
    )(cand, cand, g)


def kernel(node_embs, scorer):
    embs_p = jnp.pad(node_embs, ((0, _NP - _N), (0, 0)))
    scores = _scores_call(embs_p, scorer)
    thresh = _thresh_call(scores.reshape(_NP // 128, 128))
    cand = _compact_call(scores, thresh)
    idx = cand[1].astype(jnp.int32)
    g = _sc_gather(node_embs, idx)
    return _finalize_call(cand, g)
